# baseline (speedup 1.0000x reference)
import jax
import jax.numpy as jnp
from jax import lax
from jax.experimental import pallas as pl
from jax.experimental.pallas import tpu as pltpu
from jax.experimental.pallas import tpu_sc as plsc

NUM_POINTS = 1000000
DIMS = 64
BATCH = 16384

NC = 2
NS = 16
NW = NC * NS
BPW = BATCH // NW
QC = 128
NQ = BPW // QC
LANES = 16
HROWS = NUM_POINTS // 4


def _sc_body(lorig_hbm, rorig_hbm, lrow_hbm, rrow_hbm, loff_hbm, roff_hbm,
             lsel_hbm, rsel_hbm, embA_hbm, embB_hbm, blhs_hbm, brhs_hbm,
             out_hbm,
             lrowi_v, rrowi_v, loff_v, roff_v, lsel_v, rsel_v,
             lbidx_v, rbidx_v, lrows_v, rrows_v, lb_v, rb_v, m_v, out_v,
             sem, bsem):
    wid = lax.axis_index("s") * NC + lax.axis_index("c")
    base = wid * BPW

    pltpu.sync_copy(lrow_hbm.at[pl.ds(base, BPW)], lrowi_v)
    pltpu.sync_copy(rrow_hbm.at[pl.ds(base, BPW)], rrowi_v)
    pltpu.sync_copy(loff_hbm.at[pl.ds(base, BPW)], loff_v)
    pltpu.sync_copy(roff_hbm.at[pl.ds(base, BPW)], roff_v)
    pltpu.sync_copy(lsel_hbm.at[pl.ds(base, BPW)], lsel_v)
    pltpu.sync_copy(rsel_hbm.at[pl.ds(base, BPW)], rsel_v)
    pltpu.sync_copy(lorig_hbm.at[pl.ds(base, BPW)], lbidx_v)
    pltpu.sync_copy(rorig_hbm.at[pl.ds(base, BPW)], rbidx_v)

    bcopies = []
    for c in range(NQ):
        bcopies.append(pltpu.async_copy(
            blhs_hbm.at[lbidx_v.at[pl.ds(c * QC, QC)]],
            lb_v.at[pl.ds(c * QC, QC)], bsem))
        bcopies.append(pltpu.async_copy(
            brhs_hbm.at[rbidx_v.at[pl.ds(c * QC, QC)]],
            rb_v.at[pl.ds(c * QC, QC)], bsem))

    lane = lax.iota(jnp.int32, LANES)

    def quarter(q, carry):
        idxl = lrowi_v.at[pl.ds(q * QC, QC)]
        idxr = rrowi_v.at[pl.ds(q * QC, QC)]
        copies = [
            pltpu.async_copy(embA_hbm.at[idxl], lrows_v.at[0], sem),
            pltpu.async_copy(embB_hbm.at[idxl], lrows_v.at[1], sem),
            pltpu.async_copy(embA_hbm.at[idxr], rrows_v.at[0], sem),
            pltpu.async_copy(embB_hbm.at[idxr], rrows_v.at[1], sem),
        ]
        for cp in copies:
            cp.wait()
        for blk in range(QC // LANES):
            o = q * QC + blk * LANES
            olv = loff_v[pl.ds(o, LANES)]
            orv = roff_v[pl.ds(o, LANES)]
            slv = lsel_v[pl.ds(o, LANES)]
            srv = rsel_v[pl.ds(o, LANES)]
            for j in range(LANES):
                p = blk * LANES + j
                ol = olv[j]
                orr = orv[j]
                tl = slv[j]
                tr = srv[j]
                acc = jnp.zeros((LANES,), jnp.float32)
                for k in range(DIMS // LANES):
                    lv = lrows_v[tl, p, pl.ds(ol + k * LANES, LANES)]
                    rv = rrows_v[tr, p, pl.ds(orr + k * LANES, LANES)]
                    d = lv - rv
                    acc = acc + d * d
                m_v[pl.ds(j * LANES, LANES)] = acc
            sqv = jnp.zeros((LANES,), jnp.float32)
            for k in range(LANES):
                sqv = sqv + plsc.load_gather(m_v, [lane * LANES + k])
            out_v[pl.ds(o, LANES)] = (
                lb_v[pl.ds(o, LANES)] + rb_v[pl.ds(o, LANES)] - sqv)
        return carry

    for bc in bcopies:
        bc.wait()
    lax.fori_loop(0, NQ, quarter, 0)

    pltpu.sync_copy(out_v, out_hbm.at[pl.ds(base, BPW)])


@jax.jit
def _run(lorig, rorig, lrow, rrow, loff, roff, lsel, rsel,
         embA, embB, bias_lhs, bias_rhs):
    mesh = plsc.VectorSubcoreMesh(core_axis_name="c", subcore_axis_name="s")
    f = pl.kernel(
        _sc_body,
        out_type=jax.ShapeDtypeStruct((BATCH,), jnp.float32),
        mesh=mesh,
        compiler_params=pltpu.CompilerParams(needs_layout_passes=False),
        scratch_types=[
            pltpu.VMEM((BPW,), jnp.int32),
            pltpu.VMEM((BPW,), jnp.int32),
            pltpu.VMEM((BPW,), jnp.int32),
            pltpu.VMEM((BPW,), jnp.int32),
            pltpu.VMEM((BPW,), jnp.int32),
            pltpu.VMEM((BPW,), jnp.int32),
            pltpu.VMEM((BPW,), jnp.int32),
            pltpu.VMEM((BPW,), jnp.int32),
            pltpu.VMEM((2, QC, 2 * DIMS), jnp.float32),
            pltpu.VMEM((2, QC, 2 * DIMS), jnp.float32),
            pltpu.VMEM((BPW,), jnp.float32),
            pltpu.VMEM((BPW,), jnp.float32),
            pltpu.VMEM((LANES * LANES,), jnp.float32),
            pltpu.VMEM((BPW,), jnp.float32),
            pltpu.SemaphoreType.DMA,
            pltpu.SemaphoreType.DMA,
        ],
    )
    return f(lorig, rorig, lrow, rrow, loff, roff, lsel, rsel,
             embA, embB, bias_lhs, bias_rhs)


def kernel(input_triplet, embeddings, bias_lhs, bias_rhs):
    lorig = input_triplet[:, 0].astype(jnp.int32)
    rorig = input_triplet[:, -1].astype(jnp.int32)
    embA = embeddings[: NUM_POINTS // 2].reshape(HROWS, 2 * DIMS)
    embB = embeddings[NUM_POINTS // 2:].reshape(HROWS, 2 * DIMS)
    lrow = lorig >> 1
    rrow = rorig >> 1
    lsel = (lrow >= HROWS).astype(jnp.int32)
    rsel = (rrow >= HROWS).astype(jnp.int32)
    return _run(lorig, rorig,
                lrow - lsel * HROWS, rrow - rsel * HROWS,
                (lorig & 1) * DIMS, (rorig & 1) * DIMS,
                lsel, rsel, embA, embB, bias_lhs, bias_rhs)


# speedup vs baseline: 2.4476x; 2.4476x over previous
import jax
import jax.numpy as jnp
from jax import lax
from jax.experimental import pallas as pl
from jax.experimental.pallas import tpu as pltpu
from jax.experimental.pallas import tpu_sc as plsc

NUM_POINTS = 1000000
DIMS = 64
BATCH = 16384

NC = 2
NS = 16
NW = NC * NS
BPW = BATCH // NW
QC = 128
NQ = BPW // QC
LANES = 16


def _sc_body(lidx_hbm, ridx_hbm, emb_hbm, blhs_hbm, brhs_hbm, out_hbm,
             lidx_v, ridx_v, lrows_v, rrows_v, lb_v, rb_v, m_v, out_v,
             sem, bsem):
    wid = lax.axis_index("s") * NC + lax.axis_index("c")
    base = wid * BPW

    pltpu.sync_copy(lidx_hbm.at[pl.ds(base, BPW)], lidx_v)
    pltpu.sync_copy(ridx_hbm.at[pl.ds(base, BPW)], ridx_v)

    bcopies = []
    for c in range(NQ):
        bcopies.append(pltpu.async_copy(
            blhs_hbm.at[lidx_v.at[pl.ds(c * QC, QC)]],
            lb_v.at[pl.ds(c * QC, QC)], bsem))
        bcopies.append(pltpu.async_copy(
            brhs_hbm.at[ridx_v.at[pl.ds(c * QC, QC)]],
            rb_v.at[pl.ds(c * QC, QC)], bsem))

    lane = lax.iota(jnp.int32, LANES)

    def quarter(q, carry):
        copies = []
        for blk in range(QC // LANES):
            ilv = lidx_v[pl.ds(q * QC + blk * LANES, LANES)]
            irv = ridx_v[pl.ds(q * QC + blk * LANES, LANES)]
            for j in range(LANES):
                p = blk * LANES + j
                copies.append(pltpu.async_copy(
                    emb_hbm.at[ilv[j]], lrows_v.at[p], sem))
                copies.append(pltpu.async_copy(
                    emb_hbm.at[irv[j]], rrows_v.at[p], sem))
        for cp in copies:
            cp.wait()
        for blk in range(QC // LANES):
            for j in range(LANES):
                p = blk * LANES + j
                acc = jnp.zeros((LANES,), jnp.float32)
                for k in range(DIMS // LANES):
                    lv = lrows_v[p, pl.ds(k * LANES, LANES)]
                    rv = rrows_v[p, pl.ds(k * LANES, LANES)]
                    d = lv - rv
                    acc = acc + d * d
                m_v[pl.ds(j * LANES, LANES)] = acc
            sqv = jnp.zeros((LANES,), jnp.float32)
            for k in range(LANES):
                sqv = sqv + plsc.load_gather(m_v, [lane * LANES + k])
            o = q * QC + blk * LANES
            out_v[pl.ds(o, LANES)] = (
                lb_v[pl.ds(o, LANES)] + rb_v[pl.ds(o, LANES)] - sqv)
        return carry

    for bc in bcopies:
        bc.wait()
    lax.fori_loop(0, NQ, quarter, 0)

    pltpu.sync_copy(out_v, out_hbm.at[pl.ds(base, BPW)])


@jax.jit
def _run(lidx, ridx, emb, bias_lhs, bias_rhs):
    mesh = plsc.VectorSubcoreMesh(core_axis_name="c", subcore_axis_name="s")
    f = pl.kernel(
        _sc_body,
        out_type=jax.ShapeDtypeStruct((BATCH,), jnp.float32),
        mesh=mesh,
        compiler_params=pltpu.CompilerParams(needs_layout_passes=False),
        scratch_types=[
            pltpu.VMEM((BPW,), jnp.int32),
            pltpu.VMEM((BPW,), jnp.int32),
            pltpu.VMEM((QC, DIMS), jnp.float32),
            pltpu.VMEM((QC, DIMS), jnp.float32),
            pltpu.VMEM((BPW,), jnp.float32),
            pltpu.VMEM((BPW,), jnp.float32),
            pltpu.VMEM((LANES * LANES,), jnp.float32),
            pltpu.VMEM((BPW,), jnp.float32),
            pltpu.SemaphoreType.DMA,
            pltpu.SemaphoreType.DMA,
        ],
    )
    return f(lidx, ridx, emb, bias_lhs, bias_rhs)


def kernel(input_triplet, embeddings, bias_lhs, bias_rhs):
    lidx = input_triplet[:, 0].astype(jnp.int32)
    ridx = input_triplet[:, -1].astype(jnp.int32)
    return _run(lidx, ridx, embeddings, bias_lhs, bias_rhs)
